# baseline (speedup 1.0000x reference)
import functools

import jax
import jax.numpy as jnp
from jax import lax
from jax.experimental import pallas as pl
from jax.experimental.pallas import tpu as pltpu
from jax.experimental.pallas import tpu_sc as plsc

N = 2097152
NC = 2
NS = 16
NW = NC * NS
CHUNK = N // NW
LANES = 16
NREG = 64
SS = CHUNK // NREG
RVECS = SS // LANES
NG = NREG // LANES

_mesh = plsc.VectorSubcoreMesh(core_axis_name="c", subcore_axis_name="s")
_params = pltpu.CompilerParams(needs_layout_passes=False)


def _wid():
    return lax.axis_index("c") * NS + lax.axis_index("s")


@functools.partial(
    pl.kernel,
    out_type=jax.ShapeDtypeStruct((NW * NREG,), jnp.float32),
    mesh=_mesh,
    compiler_params=_params,
    scratch_types=[
        pltpu.VMEM((CHUNK,), jnp.float32),
        pltpu.VMEM((NREG,), jnp.float32),
    ],
)
def _region_sums(x_hbm, out_hbm, buf, totals_v):
    wid = _wid()
    pltpu.sync_copy(x_hbm.at[pl.ds(wid * CHUNK, CHUNK)], buf)
    lane = lax.iota(jnp.int32, LANES)
    mask0 = lane == 0

    def region_body(j, _):
        def ibody(t, accs):
            a0, a1, a2, a3 = accs
            base = j * SS + t * (4 * LANES)
            a0 = a0 + buf[pl.ds(base + 0 * LANES, LANES)]
            a1 = a1 + buf[pl.ds(base + 1 * LANES, LANES)]
            a2 = a2 + buf[pl.ds(base + 2 * LANES, LANES)]
            a3 = a3 + buf[pl.ds(base + 3 * LANES, LANES)]
            return (a0, a1, a2, a3)

        z = jnp.zeros((LANES,), jnp.float32)
        a0, a1, a2, a3 = lax.fori_loop(0, RVECS // 4, ibody, (z, z, z, z))
        t = jnp.sum((a0 + a1) + (a2 + a3))
        idx = jnp.full((LANES,), j, jnp.int32)
        plsc.store_scatter(totals_v, [idx], jnp.full((LANES,), t), mask=mask0)
        return 0

    lax.fori_loop(0, NREG, region_body, 0)
    pltpu.sync_copy(totals_v, out_hbm.at[pl.ds(wid * NREG, NREG)])


@functools.partial(
    pl.kernel,
    out_type=jax.ShapeDtypeStruct((N,), jnp.float32),
    mesh=_mesh,
    compiler_params=_params,
    scratch_types=[
        pltpu.VMEM((CHUNK,), jnp.float32),
        pltpu.VMEM((NW * NREG,), jnp.float32),
    ],
)
def _scan_chunks(x_hbm, tot_hbm, out_hbm, buf, tot_v):
    wid = _wid()
    pltpu.sync_copy(x_hbm.at[pl.ds(wid * CHUNK, CHUNK)], buf)
    pltpu.sync_copy(tot_hbm, tot_v)

    def gbody(t, acc):
        v = tot_v[pl.ds(t * LANES, LANES)]
        keep = (t < NG * wid).astype(jnp.float32)
        return acc + v * keep

    accg = lax.fori_loop(0, NW * NREG // LANES, gbody,
                         jnp.zeros((LANES,), jnp.float32))
    carry_g = jnp.sum(accg) - 1.0

    r = [tot_v[pl.ds(wid * NREG + g * LANES, LANES)] for g in range(NG)]
    s = [jnp.cumsum(rg) for rg in r]
    base = [carry_g] * NG
    for g in range(1, NG):
        base[g] = base[g - 1] + s[g - 1][15]
    runs = [(s[g] - r[g]) + base[g] for g in range(NG)]
    lane = lax.iota(jnp.int32, LANES)
    idxs = [(lane + g * LANES) * SS for g in range(NG)]

    @plsc.parallel_loop(0, SS, unroll=4, carry=tuple(runs) + tuple(idxs))
    def body(k, c):
        del k
        rs, ixs = list(c[:NG]), list(c[NG:])
        for g in range(NG):
            v = plsc.load_gather(buf, [ixs[g]])
            rs[g] = rs[g] + v
            plsc.store_scatter(buf, [ixs[g]], rs[g])
            ixs[g] = ixs[g] + 1
        return tuple(rs) + tuple(ixs)

    pltpu.sync_copy(buf, out_hbm.at[pl.ds(wid * CHUNK, CHUNK)])


def kernel(mask_i):
    sums = _region_sums(mask_i)
    return _scan_chunks(mask_i, sums)


# speedup vs baseline: 1.6183x; 1.6183x over previous
import functools

import jax
import jax.numpy as jnp
from jax import lax
from jax.experimental import pallas as pl
from jax.experimental.pallas import tpu as pltpu
from jax.experimental.pallas import tpu_sc as plsc

N = 2097152
NC = 2
NS = 16
NW = NC * NS
CHUNK = N // NW
LANES = 16
_U = 8
HALF = CHUNK // 2
SUB = CHUNK // 4
SUB_GROUPS = SUB // (_U * LANES)

_mesh = plsc.VectorSubcoreMesh(core_axis_name="c", subcore_axis_name="s")
_params = pltpu.CompilerParams(needs_layout_passes=False)


def _wid():
    return lax.axis_index("c") * NS + lax.axis_index("s")


@functools.partial(
    pl.kernel,
    out_type=jax.ShapeDtypeStruct((NW * LANES,), jnp.float32),
    mesh=_mesh,
    compiler_params=_params,
    scratch_types=[
        pltpu.VMEM((HALF,), jnp.float32),
        pltpu.VMEM((HALF,), jnp.float32),
        pltpu.VMEM((LANES,), jnp.float32),
        pltpu.SemaphoreType.DMA,
        pltpu.SemaphoreType.DMA,
    ],
)
def _chunk_sums(x_hbm, out_hbm, buf0, buf1, accv, sem0, sem1):
    wid = _wid()
    base = wid * CHUNK
    bufs = (buf0, buf1)
    sems = (sem0, sem1)
    copies = [
        pltpu.async_copy(x_hbm.at[pl.ds(base + h * HALF, HALF)], bufs[h], sems[h])
        for h in range(2)
    ]

    z = jnp.zeros((LANES,), jnp.float32)
    accs = (z, z, z, z)
    for h in range(2):
        copies[h].wait()
        buf = bufs[h]

        @plsc.parallel_loop(0, HALF // (_U * LANES), carry=accs)
        def hbody(g, a):
            a0, a1, a2, a3 = a
            o = g * (_U * LANES)
            a0 = a0 + buf[pl.ds(o + 0 * LANES, LANES)]
            a1 = a1 + buf[pl.ds(o + 1 * LANES, LANES)]
            a2 = a2 + buf[pl.ds(o + 2 * LANES, LANES)]
            a3 = a3 + buf[pl.ds(o + 3 * LANES, LANES)]
            a0 = a0 + buf[pl.ds(o + 4 * LANES, LANES)]
            a1 = a1 + buf[pl.ds(o + 5 * LANES, LANES)]
            a2 = a2 + buf[pl.ds(o + 6 * LANES, LANES)]
            a3 = a3 + buf[pl.ds(o + 7 * LANES, LANES)]
            return (a0, a1, a2, a3)

        accs = hbody

    a0, a1, a2, a3 = accs
    accv[...] = (a0 + a1) + (a2 + a3)
    pltpu.sync_copy(accv, out_hbm.at[pl.ds(wid * LANES, LANES)])


@functools.partial(
    pl.kernel,
    out_type=jax.ShapeDtypeStruct((N,), jnp.float32),
    mesh=_mesh,
    compiler_params=_params,
    scratch_types=[
        pltpu.VMEM((SUB,), jnp.float32),
        pltpu.VMEM((SUB,), jnp.float32),
        pltpu.VMEM((SUB,), jnp.float32),
        pltpu.VMEM((SUB,), jnp.float32),
        pltpu.VMEM((NW * LANES,), jnp.float32),
        pltpu.SemaphoreType.DMA,
        pltpu.SemaphoreType.DMA,
        pltpu.SemaphoreType.DMA,
        pltpu.SemaphoreType.DMA,
    ],
)
def _scan_chunks(x_hbm, sums_hbm, out_hbm, in0, in1, out0, out1, sums_v,
                 isem0, isem1, osem0, osem1):
    wid = _wid()
    base = wid * CHUNK
    ins = (in0, in1)
    outs = (out0, out1)
    isems = (isem0, isem1)
    osems = (osem0, osem1)

    pltpu.sync_copy(sums_hbm, sums_v)

    in_copies = [None] * 4
    out_copies = [None] * 4
    for b in range(2):
        in_copies[b] = pltpu.async_copy(
            x_hbm.at[pl.ds(base + b * SUB, SUB)], ins[b], isems[b])

    def off_body(w, acc):
        v = sums_v[pl.ds(w * LANES, LANES)]
        keep = (w < wid).astype(jnp.float32)
        return acc + v * keep

    offv = lax.fori_loop(0, NW, off_body, jnp.zeros((LANES,), jnp.float32))
    carry = jnp.sum(offv) - 1.0

    for b in range(4):
        in_copies[b].wait()
        if b >= 2:
            out_copies[b - 2].wait()
        ibuf = ins[b % 2]
        obuf = outs[b % 2]

        @plsc.parallel_loop(0, SUB_GROUPS, carry=carry)
        def body(g, c):
            o = g * (_U * LANES)
            ss = []
            ts = []
            for j in range(_U):
                v = ibuf[pl.ds(o + j * LANES, LANES)]
                s = jnp.cumsum(v)
                ss.append(s)
                ts.append(s[15])
            t01 = ts[0] + ts[1]
            t23 = ts[2] + ts[3]
            t45 = ts[4] + ts[5]
            t67 = ts[6] + ts[7]
            t03 = t01 + t23
            e = [None] * _U
            e[1] = ts[0]
            e[2] = t01
            e[3] = t01 + ts[2]
            e[4] = t03
            e[5] = t03 + ts[4]
            e[6] = t03 + t45
            e[7] = e[6] + ts[6]
            obuf[pl.ds(o, LANES)] = ss[0] + c
            for j in range(1, _U):
                obuf[pl.ds(o + j * LANES, LANES)] = ss[j] + (c + e[j])
            return c + (t03 + (t45 + t67))

        carry = body
        out_copies[b] = pltpu.async_copy(
            obuf, out_hbm.at[pl.ds(base + b * SUB, SUB)], osems[b % 2])
        if b + 2 < 4:
            in_copies[b + 2] = pltpu.async_copy(
                x_hbm.at[pl.ds(base + (b + 2) * SUB, SUB)], ins[b % 2], isems[b % 2])

    out_copies[2].wait()
    out_copies[3].wait()


def kernel(mask_i):
    sums = _chunk_sums(mask_i)
    return _scan_chunks(mask_i, sums)
